# baseline (speedup 1.0000x reference)
import functools

import jax
import jax.numpy as jnp
from jax.experimental import pallas as pl
from jax.experimental.pallas import tpu as pltpu

TOPK = 16
QB = 256
KB = 2048
NG = 8
SW = KB // NG

_NEG_INF = float("-inf")
_BIG_IDX = 2**30


def _prenorm_body(k_ref, out_ref):
    k = k_ref[...]
    ss = jnp.sum(k * k, axis=1, keepdims=True)
    out_ref[...] = k / (jnp.sqrt(ss) + 1e-12)


def _knn_body(n_total, n_kb, q_ref, k_ref, vals_ref, idx_ref,
              qn_ref, rv_ref, ri_ref, vscr_ref, m_ref):
    kb = pl.program_id(1)

    @pl.when(kb == 0)
    def _init():
        q = q_ref[...]
        ss = jnp.sum(q * q, axis=1, keepdims=True)
        qn_ref[...] = q / (jnp.sqrt(ss) + 1e-12)
        rv_ref[...] = jnp.full((QB, TOPK), _NEG_INF, jnp.float32)
        ri_ref[...] = jnp.zeros((QB, TOPK), jnp.float32)

    qn = qn_ref[...]
    k = k_ref[...]
    s = jax.lax.dot_general(qn, k, (((1,), (1,)), ((), ())),
                            preferred_element_type=jnp.float32)
    gcol_i = jax.lax.broadcasted_iota(jnp.int32, (QB, KB), 1) + kb * KB
    s = jnp.where(gcol_i < n_total, s, _NEG_INF)
    vscr_ref[...] = s
    m0 = jnp.concatenate(
        [jnp.max(s[:, p * SW:(p + 1) * SW], axis=1, keepdims=True)
         for p in range(NG)], axis=1)
    m_ref[...] = m0
    go0 = jnp.any(m0 >= rv_ref[:, TOPK - 1:TOPK])

    def body(_):
        m8 = m_ref[...]
        ent = m8 >= rv_ref[:, TOPK - 1:TOPK]
        colact = jnp.max(ent.astype(jnp.float32), axis=0, keepdims=True)
        for p in range(NG):
            act = colact[0, p] > 0.0

            @pl.when(act)
            def _proc(p=p):
                v = vscr_ref[:, p * SW:(p + 1) * SW]
                mp = m_ref[:, p:p + 1]
                gc = (jax.lax.broadcasted_iota(jnp.int32, (QB, SW), 1)
                      + (kb * KB + p * SW))
                c = jnp.min(jnp.where(v == mp, gc, _BIG_IDX), axis=1,
                            keepdims=True)
                v2 = jnp.where(gc == c, _NEG_INF, v)
                vscr_ref[:, p * SW:(p + 1) * SW] = v2
                m_ref[:, p:p + 1] = jnp.max(v2, axis=1, keepdims=True)
                cf = c.astype(jnp.float32)
                rv = rv_ref[...]
                ri = ri_ref[...]
                shv = jnp.concatenate(
                    [jnp.full((QB, 1), jnp.inf, jnp.float32),
                     rv[:, :TOPK - 1]], axis=1)
                shi = jnp.concatenate(
                    [jnp.zeros((QB, 1), jnp.float32), ri[:, :TOPK - 1]],
                    axis=1)
                kp = (rv > mp) | ((rv == mp) & (ri < cf))
                kps = (shv > mp) | ((shv == mp) & (shi < cf))
                rv_ref[...] = jnp.where(kp, rv, jnp.where(kps, mp, shv))
                ri_ref[...] = jnp.where(kp, ri, jnp.where(kps, cf, shi))

        return jnp.any(m_ref[...] >= rv_ref[:, TOPK - 1:TOPK])

    jax.lax.while_loop(lambda g: g, body, go0)

    @pl.when(kb == n_kb - 1)
    def _out():
        vals_ref[...] = rv_ref[...]
        idx_ref[...] = ri_ref[...].astype(jnp.int32)


@jax.jit
def kernel(queries, keys):
    q_n, d = queries.shape
    n = keys.shape[0]
    n_pad = pl.cdiv(n, KB) * KB
    n_kb = n_pad // KB
    n_qb = q_n // QB

    kpad = jnp.pad(keys, ((0, n_pad - n), (0, 0)))
    kn = pl.pallas_call(
        _prenorm_body,
        grid=(n_kb,),
        in_specs=[pl.BlockSpec((KB, d), lambda i: (i, 0))],
        out_specs=pl.BlockSpec((KB, d), lambda i: (i, 0)),
        out_shape=jax.ShapeDtypeStruct((n_pad, d), jnp.float32),
    )(kpad)

    vals, idx = pl.pallas_call(
        functools.partial(_knn_body, n, n_kb),
        grid=(n_qb, n_kb),
        in_specs=[
            pl.BlockSpec((QB, d), lambda qb, kb: (qb, 0)),
            pl.BlockSpec((KB, d), lambda qb, kb: (kb, 0)),
        ],
        out_specs=[
            pl.BlockSpec((QB, TOPK), lambda qb, kb: (qb, 0)),
            pl.BlockSpec((QB, TOPK), lambda qb, kb: (qb, 0)),
        ],
        out_shape=[
            jax.ShapeDtypeStruct((q_n, TOPK), jnp.float32),
            jax.ShapeDtypeStruct((q_n, TOPK), jnp.int32),
        ],
        scratch_shapes=[
            pltpu.VMEM((QB, d), jnp.float32),
            pltpu.VMEM((QB, TOPK), jnp.float32),
            pltpu.VMEM((QB, TOPK), jnp.float32),
            pltpu.VMEM((QB, KB), jnp.float32),
            pltpu.VMEM((QB, NG), jnp.float32),
        ],
    )(queries, kn)
    return vals, idx
